# baseline (speedup 1.0000x reference)
import functools

import jax
import jax.numpy as jnp
from jax import lax
from jax.experimental import pallas as pl
from jax.experimental.pallas import tpu as pltpu
from jax.experimental.pallas import tpu_sc as plsc

_TK = 2048


def _topk_body(K, TK, nsteps, q_ref, k_ref, iota_ref, idx_out, vmax_ref):
    i = pl.program_id(0)

    @pl.when(i == 0)
    def _init():
        vmax_ref[:] = jnp.full_like(vmax_ref, -jnp.inf)
        idx_out[:] = jnp.zeros_like(idx_out)

    s = lax.dot_general(
        q_ref[:], k_ref[:],
        dimension_numbers=(((1,), (1,)), ((), ())),
        preferred_element_type=jnp.float32,
    )

    def _update(s):
        tmax = jnp.max(s, axis=1, keepdims=True)
        big = jnp.iinfo(jnp.int32).max
        loc = jnp.min(jnp.where(s == tmax, iota_ref[:], big), axis=1,
                      keepdims=True)
        better = tmax > vmax_ref[:]
        idx_out[:] = jnp.where(better, i * TK + loc, idx_out[:])
        vmax_ref[:] = jnp.where(better, tmax, vmax_ref[:])

    @pl.when(i < nsteps - 1)
    def _full_tile():
        _update(s)

    @pl.when(i == nsteps - 1)
    def _tail_tile():
        _update(jnp.where(iota_ref[:] < K - i * TK, s, -jnp.inf))


def _fused_top1(queries, keys):
    B, D = queries.shape
    K = keys.shape[0]
    nsteps = (K + _TK - 1) // _TK
    iota = lax.iota(jnp.int32, _TK).reshape(1, _TK)
    idx2d = pl.pallas_call(
        functools.partial(_topk_body, K, _TK, nsteps),
        grid=(nsteps,),
        in_specs=[
            pl.BlockSpec((B, D), lambda i: (0, 0)),
            pl.BlockSpec((_TK, D), lambda i: (i, 0)),
            pl.BlockSpec((1, _TK), lambda i: (0, 0)),
        ],
        out_specs=pl.BlockSpec((B, 1), lambda i: (0, 0)),
        out_shape=jax.ShapeDtypeStruct((B, 1), jnp.int32),
        scratch_shapes=[pltpu.VMEM((B, 1), jnp.float32)],
    )(queries, keys, iota)
    return idx2d.reshape(B)


def _sc_gather(table, idx):
    V, D = table.shape
    B = idx.shape[0]
    info = plsc.get_sparse_core_info()
    NC, NS = info.num_cores, info.num_subcores
    NW = NC * NS
    b_per_w = B // NW
    mesh = plsc.VectorSubcoreMesh(core_axis_name="c", subcore_axis_name="s")

    @functools.partial(
        pl.kernel,
        mesh=mesh,
        out_type=jax.ShapeDtypeStruct((B, D), jnp.float32),
        scratch_types=[
            pltpu.VMEM((b_per_w,), jnp.int32),
            pltpu.VMEM((b_per_w, D), jnp.float32),
            pltpu.SemaphoreType.DMA,
        ],
        compiler_params=pltpu.CompilerParams(use_tc_tiling_on_sc=False),
    )
    def gather_kernel(table_hbm, idx_hbm, out_hbm, idx_v, rows_v, sem):
        wid = lax.axis_index("s") * NC + lax.axis_index("c")
        base = wid * b_per_w
        pltpu.sync_copy(idx_hbm.at[pl.ds(base, b_per_w)], idx_v)
        pltpu.async_copy(table_hbm.at[idx_v], rows_v, sem).wait()
        pltpu.sync_copy(rows_v, out_hbm.at[pl.ds(base, b_per_w)])

    return gather_kernel(table, idx)


def _mlp_body(obs_ref, ctx_ref, w1o_ref, w1c_ref, b1_ref, w2_ref, b2_ref, q_out):
    h = jnp.dot(obs_ref[:], w1o_ref[:], preferred_element_type=jnp.float32)
    h = h + jnp.dot(ctx_ref[:], w1c_ref[:], preferred_element_type=jnp.float32)
    h = jnp.maximum(h + b1_ref[:], 0.0)
    q_out[:] = jnp.dot(h, w2_ref[:], preferred_element_type=jnp.float32) + b2_ref[:]


def _mlp(obs, ctx, W1, b1, W2, b2):
    B, OBS = obs.shape
    D = ctx.shape[1]
    H = W1.shape[1]
    A = W2.shape[1]
    return pl.pallas_call(
        _mlp_body,
        out_shape=jax.ShapeDtypeStruct((B, A), jnp.float32),
    )(obs, ctx, W1[:OBS], W1[OBS:], b1.reshape(1, H), W2, b2.reshape(1, A))


def kernel(queries, keys, values, obs, W1, b1, W2, b2):
    top_idx = _fused_top1(queries, keys)
    ctx = _sc_gather(values, top_idx)
    return _mlp(obs, ctx, W1, b1, W2, b2)


# speedup vs baseline: 1.1783x; 1.1783x over previous
import functools

import jax
import jax.numpy as jnp
from jax import lax
from jax.experimental import pallas as pl
from jax.experimental.pallas import tpu as pltpu
from jax.experimental.pallas import tpu_sc as plsc

_TK = 2048


def _topk_body(K, TK, nsteps, q_ref, k_ref, iota_ref, idx_out, vmax_ref):
    i = pl.program_id(0)

    @pl.when(i == 0)
    def _init():
        vmax_ref[:] = jnp.full_like(vmax_ref, -jnp.inf)
        idx_out[:] = jnp.zeros_like(idx_out)

    s = jnp.dot(q_ref[:], k_ref[:], preferred_element_type=jnp.float32)

    def _update(s):
        tmax = jnp.max(s, axis=1, keepdims=True)
        big = jnp.iinfo(jnp.int32).max
        loc = jnp.min(jnp.where(s == tmax, iota_ref[:], big), axis=1,
                      keepdims=True)
        better = tmax > vmax_ref[:]
        idx_out[:] = jnp.where(better, i * TK + loc, idx_out[:])
        vmax_ref[:] = jnp.where(better, tmax, vmax_ref[:])

    @pl.when(i < nsteps - 1)
    def _full_tile():
        _update(s)

    @pl.when(i == nsteps - 1)
    def _tail_tile():
        _update(jnp.where(iota_ref[:] < K - i * TK, s, -jnp.inf))


def _fused_top1(queries, keys):
    B, D = queries.shape
    K = keys.shape[0]
    keys_t = keys.T
    nsteps = (K + _TK - 1) // _TK
    iota = lax.iota(jnp.int32, _TK).reshape(1, _TK)
    idx2d = pl.pallas_call(
        functools.partial(_topk_body, K, _TK, nsteps),
        grid=(nsteps,),
        in_specs=[
            pl.BlockSpec((B, D), lambda i: (0, 0)),
            pl.BlockSpec((D, _TK), lambda i: (0, i)),
            pl.BlockSpec((1, _TK), lambda i: (0, 0)),
        ],
        out_specs=pl.BlockSpec((B, 1), lambda i: (0, 0)),
        out_shape=jax.ShapeDtypeStruct((B, 1), jnp.int32),
        scratch_shapes=[pltpu.VMEM((B, 1), jnp.float32)],
    )(queries, keys_t, iota)
    return idx2d.reshape(B)


def _sc_gather(table, idx):
    V, D = table.shape
    B = idx.shape[0]
    info = plsc.get_sparse_core_info()
    NC, NS = info.num_cores, info.num_subcores
    NW = NC * NS
    b_per_w = B // NW
    mesh = plsc.VectorSubcoreMesh(core_axis_name="c", subcore_axis_name="s")

    @functools.partial(
        pl.kernel,
        mesh=mesh,
        out_type=jax.ShapeDtypeStruct((B, D), jnp.float32),
        scratch_types=[
            pltpu.VMEM((b_per_w,), jnp.int32),
            pltpu.VMEM((b_per_w, D), jnp.float32),
            pltpu.SemaphoreType.DMA,
        ],
        compiler_params=pltpu.CompilerParams(use_tc_tiling_on_sc=False),
    )
    def gather_kernel(table_hbm, idx_hbm, out_hbm, idx_v, rows_v, sem):
        wid = lax.axis_index("s") * NC + lax.axis_index("c")
        base = wid * b_per_w
        pltpu.sync_copy(idx_hbm.at[pl.ds(base, b_per_w)], idx_v)
        pltpu.async_copy(table_hbm.at[idx_v], rows_v, sem).wait()
        pltpu.sync_copy(rows_v, out_hbm.at[pl.ds(base, b_per_w)])

    return gather_kernel(table, idx)


def _mlp_body(obs_ref, ctx_ref, w1o_ref, w1c_ref, b1_ref, w2_ref, b2_ref, q_out):
    h = jnp.dot(obs_ref[:], w1o_ref[:], preferred_element_type=jnp.float32)
    h = h + jnp.dot(ctx_ref[:], w1c_ref[:], preferred_element_type=jnp.float32)
    h = jnp.maximum(h + b1_ref[:], 0.0)
    q_out[:] = jnp.dot(h, w2_ref[:], preferred_element_type=jnp.float32) + b2_ref[:]


def _mlp(obs, ctx, W1, b1, W2, b2):
    B, OBS = obs.shape
    D = ctx.shape[1]
    H = W1.shape[1]
    A = W2.shape[1]
    return pl.pallas_call(
        _mlp_body,
        out_shape=jax.ShapeDtypeStruct((B, A), jnp.float32),
    )(obs, ctx, W1[:OBS], W1[OBS:], b1.reshape(1, H), W2, b2.reshape(1, A))


def kernel(queries, keys, values, obs, W1, b1, W2, b2):
    top_idx = _fused_top1(queries, keys)
    ctx = _sc_gather(values, top_idx)
    return _mlp(obs, ctx, W1, b1, W2, b2)


# speedup vs baseline: 1.3068x; 1.1091x over previous
import functools

import jax
import jax.numpy as jnp
from jax import lax
from jax.experimental import pallas as pl
from jax.experimental.pallas import tpu as pltpu
from jax.experimental.pallas import tpu_sc as plsc

_TK = 2048


def _topk_body(K, TK, nsteps, q_ref, k_ref, iota_ref, idx_out, vmax_ref):
    i = pl.program_id(0)

    @pl.when(i == 0)
    def _init():
        vmax_ref[:] = jnp.full_like(vmax_ref, -jnp.inf)
        idx_out[:] = jnp.zeros_like(idx_out)

    s = jnp.dot(q_ref[:], k_ref[:], preferred_element_type=jnp.float32)

    def _update(s):
        tmax = jnp.max(s, axis=1, keepdims=True)
        loc = jnp.min(jnp.where(s == tmax, iota_ref[:], jnp.float32(2**30)),
                      axis=1, keepdims=True)
        better = tmax > vmax_ref[:]
        idx_out[:] = jnp.where(better, i * TK + loc.astype(jnp.int32),
                               idx_out[:])
        vmax_ref[:] = jnp.where(better, tmax, vmax_ref[:])

    @pl.when(i < nsteps - 1)
    def _full_tile():
        _update(s)

    @pl.when(i == nsteps - 1)
    def _tail_tile():
        tail = (K - i * TK).astype(jnp.float32)
        _update(jnp.where(iota_ref[:] < tail, s, -jnp.inf))


def _fused_top1(queries, keys):
    B, D = queries.shape
    K = keys.shape[0]
    keys_t = keys.T
    nsteps = (K + _TK - 1) // _TK
    iota = lax.iota(jnp.float32, _TK).reshape(1, _TK)
    idx2d = pl.pallas_call(
        functools.partial(_topk_body, K, _TK, nsteps),
        grid=(nsteps,),
        in_specs=[
            pl.BlockSpec((B, D), lambda i: (0, 0)),
            pl.BlockSpec((D, _TK), lambda i: (0, i)),
            pl.BlockSpec((1, _TK), lambda i: (0, 0)),
        ],
        out_specs=pl.BlockSpec((B, 1), lambda i: (0, 0)),
        out_shape=jax.ShapeDtypeStruct((B, 1), jnp.int32),
        scratch_shapes=[pltpu.VMEM((B, 1), jnp.float32)],
    )(queries, keys_t, iota)
    return idx2d.reshape(B)


def _sc_gather(table, idx):
    V, D = table.shape
    B = idx.shape[0]
    info = plsc.get_sparse_core_info()
    NC, NS = info.num_cores, info.num_subcores
    NW = NC * NS
    b_per_w = B // NW
    mesh = plsc.VectorSubcoreMesh(core_axis_name="c", subcore_axis_name="s")

    @functools.partial(
        pl.kernel,
        mesh=mesh,
        out_type=jax.ShapeDtypeStruct((B, D), jnp.float32),
        scratch_types=[
            pltpu.VMEM((b_per_w,), jnp.int32),
            pltpu.VMEM((b_per_w, D), jnp.float32),
            pltpu.SemaphoreType.DMA,
        ],
        compiler_params=pltpu.CompilerParams(use_tc_tiling_on_sc=False),
    )
    def gather_kernel(table_hbm, idx_hbm, out_hbm, idx_v, rows_v, sem):
        wid = lax.axis_index("s") * NC + lax.axis_index("c")
        base = wid * b_per_w
        pltpu.sync_copy(idx_hbm.at[pl.ds(base, b_per_w)], idx_v)
        pltpu.async_copy(table_hbm.at[idx_v], rows_v, sem).wait()
        pltpu.sync_copy(rows_v, out_hbm.at[pl.ds(base, b_per_w)])

    return gather_kernel(table, idx)


def _mlp_body(obs_ref, ctx_ref, w1o_ref, w1c_ref, b1_ref, w2_ref, b2_ref, q_out):
    h = jnp.dot(obs_ref[:], w1o_ref[:], preferred_element_type=jnp.float32)
    h = h + jnp.dot(ctx_ref[:], w1c_ref[:], preferred_element_type=jnp.float32)
    h = jnp.maximum(h + b1_ref[:], 0.0)
    q_out[:] = jnp.dot(h, w2_ref[:], preferred_element_type=jnp.float32) + b2_ref[:]


def _mlp(obs, ctx, W1, b1, W2, b2):
    B, OBS = obs.shape
    D = ctx.shape[1]
    H = W1.shape[1]
    A = W2.shape[1]
    return pl.pallas_call(
        _mlp_body,
        out_shape=jax.ShapeDtypeStruct((B, A), jnp.float32),
    )(obs, ctx, W1[:OBS], W1[OBS:], b1.reshape(1, H), W2, b2.reshape(1, A))


def kernel(queries, keys, values, obs, W1, b1, W2, b2):
    top_idx = _fused_top1(queries, keys)
    ctx = _sc_gather(values, top_idx)
    return _mlp(obs, ctx, W1, b1, W2, b2)


# speedup vs baseline: 1.4053x; 1.0754x over previous
import functools

import jax
import jax.numpy as jnp
from jax import lax
from jax.experimental import pallas as pl
from jax.experimental.pallas import tpu as pltpu
from jax.experimental.pallas import tpu_sc as plsc

_TK = 2048


def _topk_body(K, TK, nsteps, q_ref, ka_ref, kb_ref, iota_ref, idx_out,
               vmax_ref, sa_ref, sb_ref):
    g = pl.program_id(0)

    @pl.when(g == 0)
    def _init():
        vmax_ref[:] = jnp.full_like(vmax_ref, -jnp.inf)
        idx_out[:] = jnp.zeros_like(idx_out)

    def _update(j, s, guard):
        tmax = jnp.max(s, axis=1, keepdims=True)
        loc = jnp.min(jnp.where(s == tmax, iota_ref[:], jnp.float32(2**30)),
                      axis=1, keepdims=True)
        better = jnp.logical_and(tmax > vmax_ref[:], guard)
        idx_out[:] = jnp.where(better, j * TK + loc.astype(jnp.int32),
                               idx_out[:])
        vmax_ref[:] = jnp.where(better, tmax, vmax_ref[:])

    ja = 2 * g - 1
    jb = 2 * g

    sa_ref[:] = jnp.dot(q_ref[:], ka_ref[:], preferred_element_type=jnp.float32)
    _update(ja, sb_ref[:], jnp.logical_and(ja >= 0, ja < nsteps - 1))
    sb_ref[:] = jnp.dot(q_ref[:], kb_ref[:], preferred_element_type=jnp.float32)
    _update(jb, sa_ref[:], jb < nsteps - 1)

    @pl.when(g == (nsteps + 1) // 2)
    def _tail_tile():
        tail = jnp.float32(K - (nsteps - 1) * TK)
        s = jnp.where(iota_ref[:] < tail, sa_ref[:], -jnp.inf)
        _update(nsteps - 1, s, True)


def _fused_top1(queries, keys):
    B, D = queries.shape
    K = keys.shape[0]
    keys_t = keys.T
    nsteps = (K + _TK - 1) // _TK
    ngrid = (nsteps + 1) // 2 + 1
    last = nsteps - 1
    iota = lax.iota(jnp.float32, _TK).reshape(1, _TK)
    idx2d = pl.pallas_call(
        functools.partial(_topk_body, K, _TK, nsteps),
        grid=(ngrid,),
        in_specs=[
            pl.BlockSpec((B, D), lambda g: (0, 0)),
            pl.BlockSpec((D, _TK), lambda g: (0, jnp.minimum(2 * g, last))),
            pl.BlockSpec((D, _TK),
                         lambda g: (0, jnp.minimum(2 * g + 1, last))),
            pl.BlockSpec((1, _TK), lambda g: (0, 0)),
        ],
        out_specs=pl.BlockSpec((B, 1), lambda g: (0, 0)),
        out_shape=jax.ShapeDtypeStruct((B, 1), jnp.int32),
        scratch_shapes=[pltpu.VMEM((B, 1), jnp.float32),
                        pltpu.VMEM((B, _TK), jnp.float32),
                        pltpu.VMEM((B, _TK), jnp.float32)],
    )(queries, keys_t, keys_t, iota)
    return idx2d.reshape(B)


def _sc_gather(table, idx):
    V, D = table.shape
    B = idx.shape[0]
    info = plsc.get_sparse_core_info()
    NC, NS = info.num_cores, info.num_subcores
    NW = NC * NS
    b_per_w = B // NW
    mesh = plsc.VectorSubcoreMesh(core_axis_name="c", subcore_axis_name="s")

    @functools.partial(
        pl.kernel,
        mesh=mesh,
        out_type=jax.ShapeDtypeStruct((B, D), jnp.float32),
        scratch_types=[
            pltpu.VMEM((b_per_w,), jnp.int32),
            pltpu.VMEM((b_per_w, D), jnp.float32),
            pltpu.SemaphoreType.DMA,
        ],
        compiler_params=pltpu.CompilerParams(use_tc_tiling_on_sc=False),
    )
    def gather_kernel(table_hbm, idx_hbm, out_hbm, idx_v, rows_v, sem):
        wid = lax.axis_index("s") * NC + lax.axis_index("c")
        base = wid * b_per_w
        pltpu.sync_copy(idx_hbm.at[pl.ds(base, b_per_w)], idx_v)
        pltpu.async_copy(table_hbm.at[idx_v], rows_v, sem).wait()
        pltpu.sync_copy(rows_v, out_hbm.at[pl.ds(base, b_per_w)])

    return gather_kernel(table, idx)


def _mlp_body(obs_ref, ctx_ref, w1o_ref, w1c_ref, b1_ref, w2_ref, b2_ref, q_out):
    h = jnp.dot(obs_ref[:], w1o_ref[:], preferred_element_type=jnp.float32)
    h = h + jnp.dot(ctx_ref[:], w1c_ref[:], preferred_element_type=jnp.float32)
    h = jnp.maximum(h + b1_ref[:], 0.0)
    q_out[:] = jnp.dot(h, w2_ref[:], preferred_element_type=jnp.float32) + b2_ref[:]


def _mlp(obs, ctx, W1, b1, W2, b2):
    B, OBS = obs.shape
    D = ctx.shape[1]
    H = W1.shape[1]
    A = W2.shape[1]
    return pl.pallas_call(
        _mlp_body,
        out_shape=jax.ShapeDtypeStruct((B, A), jnp.float32),
    )(obs, ctx, W1[:OBS], W1[OBS:], b1.reshape(1, H), W2, b2.reshape(1, A))


def kernel(queries, keys, values, obs, W1, b1, W2, b2):
    top_idx = _fused_top1(queries, keys)
    ctx = _sc_gather(values, top_idx)
    return _mlp(obs, ctx, W1, b1, W2, b2)


# speedup vs baseline: 1.4127x; 1.0052x over previous
import functools

import jax
import jax.numpy as jnp
from jax import lax
from jax.experimental import pallas as pl
from jax.experimental.pallas import tpu as pltpu
from jax.experimental.pallas import tpu_sc as plsc

_TK = 2048


def _topk_body(K, TK, nsteps, q_ref, ka_ref, kb_ref, iota_ref, idx_out,
               vmax_ref, sa_ref, sb_ref):
    g = pl.program_id(0)

    @pl.when(g == 0)
    def _init():
        vmax_ref[:] = jnp.full_like(vmax_ref, -jnp.inf)
        idx_out[:] = jnp.zeros_like(idx_out)

    def _update(j, s, guard):
        tmax = jnp.max(s, axis=1, keepdims=True)
        loc = jnp.min(jnp.where(s == tmax, iota_ref[:], jnp.float32(2**30)),
                      axis=1, keepdims=True)
        better = jnp.logical_and(tmax > vmax_ref[:], guard)
        idx_out[:] = jnp.where(better, j * TK + loc.astype(jnp.int32),
                               idx_out[:])
        vmax_ref[:] = jnp.where(better, tmax, vmax_ref[:])

    ja = 2 * g - 1
    jb = 2 * g

    sa_ref[:] = jnp.dot(q_ref[:], ka_ref[:], preferred_element_type=jnp.float32)
    _update(ja, sb_ref[:], jnp.logical_and(ja >= 0, ja < nsteps - 1))
    sb_ref[:] = jnp.dot(q_ref[:], kb_ref[:], preferred_element_type=jnp.float32)
    _update(jb, sa_ref[:], jb < nsteps - 1)

    @pl.when(g == (nsteps + 1) // 2)
    def _tail_tile():
        tail = jnp.float32(K - (nsteps - 1) * TK)
        s = jnp.where(iota_ref[:] < tail, sa_ref[:], -jnp.inf)
        _update(nsteps - 1, s, True)


def _fused_top1(queries, keys):
    B, D = queries.shape
    K = keys.shape[0]
    keys_t = keys.T
    nsteps = (K + _TK - 1) // _TK
    ngrid = (nsteps + 1) // 2 + 1
    last = nsteps - 1
    iota = lax.iota(jnp.float32, _TK).reshape(1, _TK)
    idx2d = pl.pallas_call(
        functools.partial(_topk_body, K, _TK, nsteps),
        grid=(ngrid,),
        in_specs=[
            pl.BlockSpec((B, D), lambda g: (0, 0)),
            pl.BlockSpec((D, _TK), lambda g: (0, jnp.minimum(2 * g, last))),
            pl.BlockSpec((D, _TK),
                         lambda g: (0, jnp.minimum(2 * g + 1, last))),
            pl.BlockSpec((1, _TK), lambda g: (0, 0)),
        ],
        out_specs=pl.BlockSpec((B, 1), lambda g: (0, 0)),
        out_shape=jax.ShapeDtypeStruct((B, 1), jnp.int32),
        scratch_shapes=[pltpu.VMEM((B, 1), jnp.float32),
                        pltpu.VMEM((B, _TK), jnp.float32),
                        pltpu.VMEM((B, _TK), jnp.float32)],
    )(queries, keys_t, keys_t, iota)
    return idx2d


def _sc_gather(table128, idx):
    V4, D4 = table128.shape
    B = idx.shape[0]
    info = plsc.get_sparse_core_info()
    NC, NS, L = info.num_cores, info.num_subcores, info.num_lanes
    NW = NC * NS
    b_per_w = B // NW
    mesh = plsc.VectorSubcoreMesh(core_axis_name="c", subcore_axis_name="s")

    @functools.partial(
        pl.kernel,
        mesh=mesh,
        out_type=jax.ShapeDtypeStruct((B, D4), jnp.float32),
        scratch_types=[
            pltpu.VMEM((b_per_w,), jnp.int32),
            pltpu.VMEM((b_per_w,), jnp.int32),
            pltpu.VMEM((b_per_w, D4), jnp.float32),
            pltpu.SemaphoreType.DMA,
        ],
    )
    def gather_kernel(table_hbm, idx_hbm, out_hbm, idx_v, idx4_v, rows_v, sem):
        wid = lax.axis_index("s") * NC + lax.axis_index("c")
        base = wid * b_per_w
        pltpu.sync_copy(idx_hbm.at[pl.ds(base, b_per_w)], idx_v)
        for c in range(b_per_w // L):
            sl = pl.ds(c * L, L)
            idx4_v[sl] = lax.shift_right_logical(idx_v[sl], 2)
        pltpu.async_copy(table_hbm.at[idx4_v], rows_v, sem).wait()
        pltpu.sync_copy(rows_v, out_hbm.at[pl.ds(base, b_per_w)])

    return gather_kernel(table128, idx)


def _mlp_body(D, obs_ref, ctx4_ref, idx_ref, w1o_ref, w1c_ref, b1_ref, w2_ref,
              b2_ref, q_out):
    sub = jnp.bitwise_and(idx_ref[:], 3)
    ctx = jnp.zeros((ctx4_ref.shape[0], D), jnp.float32)
    for r in range(4):
        ctx = jnp.where(sub == r, ctx4_ref[:, r * D:(r + 1) * D], ctx)
    h = jnp.dot(obs_ref[:], w1o_ref[:], preferred_element_type=jnp.float32)
    h = h + jnp.dot(ctx, w1c_ref[:], preferred_element_type=jnp.float32)
    h = jnp.maximum(h + b1_ref[:], 0.0)
    q_out[:] = jnp.dot(h, w2_ref[:], preferred_element_type=jnp.float32) + b2_ref[:]


def _mlp(obs, ctx4, idx2d, W1, b1, W2, b2):
    B, OBS = obs.shape
    D = W1.shape[0] - OBS
    H = W1.shape[1]
    A = W2.shape[1]
    return pl.pallas_call(
        functools.partial(_mlp_body, D),
        out_shape=jax.ShapeDtypeStruct((B, A), jnp.float32),
    )(obs, ctx4, idx2d, W1[:OBS], W1[OBS:], b1.reshape(1, H), W2,
      b2.reshape(1, A))


def kernel(queries, keys, values, obs, W1, b1, W2, b2):
    idx2d = _fused_top1(queries, keys)
    ctx4 = _sc_gather(values.reshape(-1, 4 * values.shape[1]), idx2d.reshape(-1))
    return _mlp(obs, ctx4, idx2d, W1, b1, W2, b2)
